# Initial kernel scaffold; baseline (speedup 1.0000x reference)
import jax
import jax.numpy as jnp
from jax.experimental import pallas as pl


def kernel(pred, target):
    raise NotImplementedError("write your pallas kernel here")


# speedup vs baseline: 38.0327x; 38.0327x over previous
import functools

import jax
import jax.numpy as jnp
import numpy as np
from jax.experimental import pallas as pl
from jax.experimental.pallas import tpu as pltpu

_BINS = 10
_EDGES = np.log(np.arange(1, _BINS, dtype=np.float64) / _BINS
                / (1.0 - np.arange(1, _BINS, dtype=np.float64) / _BINS)
                ).astype(np.float32)

_LANES = 128
_SUB = 1250


def _ghmc_kernel(pred_ref, tgt_ref, out_ref, acc_ref, *, nsteps, total):
    step = pl.program_id(0)

    @pl.when(step == 0)
    def _init():
        acc_ref[...] = jnp.zeros_like(acc_ref)

    p = pred_ref[0]
    t = tgt_ref[0].astype(jnp.float32)
    q = p - 2.0 * p * t
    bce = jnp.maximum(q, 0.0) + jnp.log1p(jnp.exp(-jnp.abs(q)))

    rows = []
    for i in range(9):
        m = q >= _EDGES[i]
        rows.append(acc_ref[i] + jnp.sum(m.astype(jnp.float32), axis=0))
    for i in range(9):
        m = q >= _EDGES[i]
        rows.append(acc_ref[9 + i] + jnp.sum(jnp.where(m, bce, 0.0), axis=0))
    rows.append(acc_ref[18] + jnp.sum(bce, axis=0))
    acc_ref[...] = jnp.stack(rows, axis=0)

    @pl.when(step == nsteps - 1)
    def _finish():
        c = [jnp.float32(total)]
        s = [jnp.sum(acc_ref[18])]
        for i in range(9):
            c.append(jnp.sum(acc_ref[i]))
            s.append(jnp.sum(acc_ref[9 + i]))
        c.append(jnp.float32(0.0))
        s.append(jnp.float32(0.0))
        counts = [c[b] - c[b + 1] for b in range(_BINS)]
        sums = [s[b] - s[b + 1] for b in range(_BINS)]
        n = sum((cb > 0.0).astype(jnp.float32) for cb in counts)
        loss = jnp.float32(0.0)
        for b in range(_BINS):
            loss += jnp.where(
                counts[b] > 0.0,
                sums[b] / (jnp.maximum(counts[b], 1.0) * n), 0.0)
        out_ref[0, 0] = loss


def kernel(pred, target):
    total = pred.shape[0] * pred.shape[1]
    assert total % (_SUB * _LANES) == 0
    nsteps = total // (_SUB * _LANES)
    p3 = pred.reshape(nsteps, _SUB, _LANES)
    t3 = target.reshape(nsteps, _SUB, _LANES)
    out = pl.pallas_call(
        functools.partial(_ghmc_kernel, nsteps=nsteps, total=float(total)),
        grid=(nsteps,),
        in_specs=[
            pl.BlockSpec((1, _SUB, _LANES), lambda i: (i, 0, 0)),
            pl.BlockSpec((1, _SUB, _LANES), lambda i: (i, 0, 0)),
        ],
        out_specs=pl.BlockSpec(memory_space=pltpu.SMEM),
        out_shape=jax.ShapeDtypeStruct((1, 1), jnp.float32),
        scratch_shapes=[pltpu.VMEM((19, _LANES), jnp.float32)],
    )(p3, t3)
    return out[0, 0]


# speedup vs baseline: 61.3617x; 1.6134x over previous
import functools

import jax
import jax.numpy as jnp
import numpy as np
from jax.experimental import pallas as pl
from jax.experimental.pallas import tpu as pltpu

_BINS = 10
_EDGES = np.log(np.arange(1, _BINS, dtype=np.float64) / _BINS
                / (1.0 - np.arange(1, _BINS, dtype=np.float64) / _BINS)
                ).astype(np.float32)

_ROWS = 2000


def _ghmc_kernel(pred_ref, tgt_ref, out_ref, acc_ref, *, nsteps, total):
    step = pl.program_id(0)

    @pl.when(step == 0)
    def _init():
        acc_ref[...] = jnp.zeros_like(acc_ref)

    p = pred_ref[...]
    ti = tgt_ref[...]
    q = jax.lax.bitcast_convert_type(
        jax.lax.bitcast_convert_type(p, jnp.int32) ^ (ti << 31), jnp.float32)
    bce = jnp.maximum(q, 0.0) + jnp.log1p(jnp.exp(-jnp.abs(p)))

    rows_c = []
    rows_s = []
    for i in range(9):
        m = q >= _EDGES[i]
        rows_c.append(acc_ref[i] + jnp.sum(m.astype(jnp.float32), axis=0))
        rows_s.append(acc_ref[9 + i] + jnp.sum(jnp.where(m, bce, 0.0), axis=0))
    rows = rows_c + rows_s + [acc_ref[18] + jnp.sum(bce, axis=0)]
    acc_ref[...] = jnp.stack(rows, axis=0)

    @pl.when(step == nsteps - 1)
    def _finish():
        c = [jnp.float32(total)]
        s = [jnp.sum(acc_ref[18])]
        for i in range(9):
            c.append(jnp.sum(acc_ref[i]))
            s.append(jnp.sum(acc_ref[9 + i]))
        c.append(jnp.float32(0.0))
        s.append(jnp.float32(0.0))
        counts = [c[b] - c[b + 1] for b in range(_BINS)]
        sums = [s[b] - s[b + 1] for b in range(_BINS)]
        n = sum((cb > 0.0).astype(jnp.float32) for cb in counts)
        loss = jnp.float32(0.0)
        for b in range(_BINS):
            loss += jnp.where(
                counts[b] > 0.0,
                sums[b] / (jnp.maximum(counts[b], 1.0) * n), 0.0)
        out_ref[0, 0] = loss


def kernel(pred, target):
    n_rows, n_cols = pred.shape
    assert n_rows % _ROWS == 0
    nsteps = n_rows // _ROWS
    out = pl.pallas_call(
        functools.partial(_ghmc_kernel, nsteps=nsteps,
                          total=float(n_rows * n_cols)),
        grid=(nsteps,),
        in_specs=[
            pl.BlockSpec((_ROWS, n_cols), lambda i: (i, 0)),
            pl.BlockSpec((_ROWS, n_cols), lambda i: (i, 0)),
        ],
        out_specs=pl.BlockSpec(memory_space=pltpu.SMEM),
        out_shape=jax.ShapeDtypeStruct((1, 1), jnp.float32),
        scratch_shapes=[pltpu.VMEM((19, n_cols), jnp.float32)],
    )(pred, target)
    return out[0, 0]
